# baseline (speedup 1.0000x reference)
import functools

import jax
import jax.numpy as jnp
from jax import lax
from jax.experimental import pallas as pl
from jax.experimental.pallas import tpu as pltpu
from jax.experimental.pallas import tpu_sc as plsc

_N = 20000
_C = 81
_D = 5
_W = 405
_NC = 2
_NS = 16
_NW = _NC * _NS
_CHUNK = 640
_NPAD = _NW * _CHUNK
_L = 16
_GCHUNK = 128
_NBLK = _N * _W // 8


def _body(tab_hbm, labels_hbm, out_hbm, idxa_v, idxb_v, off_v, blka_v,
          blkb_v, rows_v, sem):
    wid = lax.axis_index("s") * _NC + lax.axis_index("c")
    base = wid * _CHUNK

    pltpu.sync_copy(labels_hbm.at[pl.ds(base, _CHUNK)], off_v)

    lane = lax.iota(jnp.int32, _L)

    def idx_body(g, _):
        sl = pl.ds(g * _L, _L)
        lab = off_v[sl]
        row = base + g * _L + lane
        w = row * _W + lab * _D
        blk = jnp.minimum(lax.shift_right_logical(w, 3), _NBLK - 1)
        idxa_v[sl] = blk
        idxb_v[sl] = jnp.minimum(blk + 1, _NBLK - 1)
        off_v[sl] = jnp.bitwise_and(w, 7)
        return 0

    lax.fori_loop(0, _CHUNK // _L, idx_body, 0)

    copies = []
    for k in range(_CHUNK // _GCHUNK):
        gsl = pl.ds(k * _GCHUNK, _GCHUNK)
        copies.append(
            pltpu.async_copy(tab_hbm.at[idxa_v.at[gsl]], blka_v.at[gsl], sem)
        )
        copies.append(
            pltpu.async_copy(tab_hbm.at[idxb_v.at[gsl]], blkb_v.at[gsl], sem)
        )
    for c in copies:
        c.wait()

    def ext_body(g, _):
        o = off_v[pl.ds(g * _L, _L)]
        ridx = g * _L + lane
        for j in range(_D):
            pos = o + j
            in_a = pos < 8
            a = plsc.load_gather(blka_v, [ridx, jnp.minimum(pos, 7)])
            b = plsc.load_gather(blkb_v, [ridx, jnp.maximum(pos - 8, 0)])
            val = jnp.where(in_a, a, b)
            plsc.store_scatter(
                rows_v, [ridx, jnp.full((_L,), j, jnp.int32)], val
            )
        return 0

    lax.fori_loop(0, _CHUNK // _L, ext_body, 0)

    pltpu.sync_copy(rows_v, out_hbm.at[pl.ds(base, _CHUNK)])


@jax.jit
def _run(tab, labels_padded):
    mesh = plsc.VectorSubcoreMesh(core_axis_name="c", subcore_axis_name="s")
    f = functools.partial(
        pl.kernel,
        out_type=jax.ShapeDtypeStruct((_NPAD, _D), jnp.float32),
        mesh=mesh,
        scratch_types=[
            pltpu.VMEM((_CHUNK,), jnp.int32),
            pltpu.VMEM((_CHUNK,), jnp.int32),
            pltpu.VMEM((_CHUNK,), jnp.int32),
            pltpu.VMEM((_CHUNK, 8), jnp.float32),
            pltpu.VMEM((_CHUNK, 8), jnp.float32),
            pltpu.VMEM((_CHUNK, _D), jnp.float32),
            pltpu.SemaphoreType.DMA,
        ],
        compiler_params=pltpu.CompilerParams(
            use_tc_tiling_on_sc=False, needs_layout_passes=False
        ),
    )(_body)
    return f(tab, labels_padded)


def kernel(pose_pred, labels):
    N, _ = pose_pred.shape
    tab = pose_pred.reshape(_NBLK, 8)
    labels32 = labels.astype(jnp.int32)
    labels_padded = jnp.pad(labels32, (0, _NPAD - N))
    out = _run(tab, labels_padded)
    return out[:N]


# speedup vs baseline: 6.5156x; 6.5156x over previous
import functools

import jax
import jax.numpy as jnp
from jax import lax
from jax.experimental import pallas as pl
from jax.experimental.pallas import tpu as pltpu
from jax.experimental.pallas import tpu_sc as plsc

_N = 20000
_C = 81
_D = 5
_J = 405
_NBANDS = 51
_NC = 2
_NS = 16
_NW = _NC * _NS
_L = 16
_CW = 128
_NCHUNK = 157
_NPAD = _NCHUNK * _CW
_PASSES = 5


def _body(tabt_hbm, labels_hbm, out_hbm, bands_v, lab_v, rows_v, sem):
    wid = lax.axis_index("s") * _NC + lax.axis_index("c")
    lane = lax.iota(jnp.int32, _L)

    pltpu.sync_copy(labels_hbm, lab_v)

    def pass_body(p, _):
        chunk = p * _NW + wid

        @pl.when(chunk < _NCHUNK)
        def _():
            i0 = pl.multiple_of(chunk * _CW, _CW)

            copies = []
            for b in range(_NBANDS):
                span = min(8, _J - b * 8)
                copies.append(
                    pltpu.async_copy(
                        tabt_hbm.at[pl.ds(b * 8, span), pl.ds(i0, _CW)],
                        bands_v.at[b, pl.ds(0, span)],
                        sem,
                    )
                )
            for c in copies:
                c.wait()

            for g in range(_CW // _L):
                ilocal = g * _L + lane
                lab = lab_v[pl.ds(i0 + g * _L, _L)]
                j0 = jnp.clip(lab, 0, _C - 1) * _D
                band = lax.shift_right_logical(j0, 3)
                woff = jnp.bitwise_and(j0, 7)
                for j in range(_D):
                    p_ = woff + j
                    bandsel = band + lax.shift_right_logical(p_, 3)
                    wordsel = jnp.bitwise_and(p_, 7)
                    val = plsc.load_gather(bands_v, [bandsel, wordsel, ilocal])
                    plsc.store_scatter(
                        rows_v, [jnp.full((_L,), j, jnp.int32), ilocal], val
                    )

            pltpu.sync_copy(rows_v, out_hbm.at[:, pl.ds(i0, _CW)])

        return 0

    lax.fori_loop(0, _PASSES, pass_body, 0)


@jax.jit
def _run(tabt, labels32):
    mesh = plsc.VectorSubcoreMesh(core_axis_name="c", subcore_axis_name="s")
    f = functools.partial(
        pl.kernel,
        out_type=jax.ShapeDtypeStruct((_D, _NPAD), jnp.float32),
        mesh=mesh,
        scratch_types=[
            pltpu.VMEM((_NBANDS, 8, _CW), jnp.float32),
            pltpu.VMEM((_N,), jnp.int32),
            pltpu.VMEM((_D, _CW), jnp.float32),
            pltpu.SemaphoreType.DMA,
        ],
        compiler_params=pltpu.CompilerParams(
            use_tc_tiling_on_sc=True,
            disable_bounds_checks=True,
            needs_layout_passes=False,
        ),
    )(_body)
    return f(tabt, labels32)


def kernel(pose_pred, labels):
    out_t = _run(pose_pred.T, labels.astype(jnp.int32))
    return out_t.T[:_N]
